# Initial kernel scaffold; baseline (speedup 1.0000x reference)
import jax
import jax.numpy as jnp
from jax.experimental import pallas as pl


def kernel(X, edge_index, W0, b0, W1, b1, W2, b2, W3, b3, W4, b4):
    raise NotImplementedError("write your pallas kernel here")


# speedup vs baseline: 9.6454x; 9.6454x over previous
import functools

import jax
import jax.numpy as jnp
from jax import lax
from jax.experimental import pallas as pl
from jax.experimental.pallas import tpu as pltpu
from jax.experimental.pallas import tpu_sc as plsc

N = 10000
N_PAD = 10240
E = 320000
IN_LEN = 128
HID = 64

NC = 2
NS = 16
NW = NC * NS
CHUNK = 128
E_ROWS = 2560
E_PAD = E_ROWS * CHUNK
ROWS_PER_W = E_ROWS // NW
TILE_ROWS = N_PAD // NS
DUMMY = N

BLK = 256
GRID = N_PAD // BLK

_mesh = plsc.VectorSubcoreMesh(core_axis_name="c", subcore_axis_name="s")
_sc_params = pltpu.CompilerParams(use_tc_tiling_on_sc=False)


@functools.partial(
    pl.kernel,
    out_type=jax.ShapeDtypeStruct((NC * N_PAD, 16), jnp.float32),
    mesh=_mesh,
    scratch_types=[
        pltpu.VMEM((ROWS_PER_W, CHUNK), jnp.int32),
        pltpu.VMEM((CHUNK, 16), jnp.float32),
        pltpu.VMEM_SHARED((N_PAD, 16), jnp.float32),
    ],
    compiler_params=_sc_params,
)
def _sc_degree(dst_hbm, ones_hbm, zeros_hbm, deg_hbm, dst_v, ones_v, shared_deg):
    c = lax.axis_index("c")
    s = lax.axis_index("s")
    w = c * NS + s
    pltpu.sync_copy(zeros_hbm, shared_deg.at[pl.ds(s * TILE_ROWS, TILE_ROWS)])
    pltpu.sync_copy(ones_hbm, ones_v)
    pltpu.sync_copy(dst_hbm.at[pl.ds(w * ROWS_PER_W, ROWS_PER_W)], dst_v)
    plsc.subcore_barrier()

    def body(j, carry):
        pltpu.sync_copy(ones_v, shared_deg.at[dst_v.at[j]], add=True)
        return carry

    lax.fori_loop(0, ROWS_PER_W, body, 0)
    plsc.subcore_barrier()
    pltpu.sync_copy(
        shared_deg.at[pl.ds(s * TILE_ROWS, TILE_ROWS)],
        deg_hbm.at[pl.ds(c * N_PAD + s * TILE_ROWS, TILE_ROWS)],
    )


@functools.partial(
    pl.kernel,
    out_type=jax.ShapeDtypeStruct((NC * N_PAD, HID), jnp.float32),
    mesh=_mesh,
    scratch_types=[
        pltpu.VMEM((ROWS_PER_W, CHUNK), jnp.int32),
        pltpu.VMEM((ROWS_PER_W, CHUNK), jnp.int32),
        pltpu.VMEM((CHUNK, HID), jnp.float32),
        pltpu.VMEM_SHARED((N_PAD, HID), jnp.float32),
        pltpu.SemaphoreType.DMA,
    ],
    compiler_params=_sc_params,
)
def _sc_aggregate(htn_hbm, src_hbm, dst_hbm, zeros_hbm, agg_hbm,
                  src_v, dst_v, rows_v, shared_agg, sem):
    c = lax.axis_index("c")
    s = lax.axis_index("s")
    w = c * NS + s
    pltpu.sync_copy(zeros_hbm, shared_agg.at[pl.ds(s * TILE_ROWS, TILE_ROWS)])
    pltpu.sync_copy(src_hbm.at[pl.ds(w * ROWS_PER_W, ROWS_PER_W)], src_v)
    pltpu.sync_copy(dst_hbm.at[pl.ds(w * ROWS_PER_W, ROWS_PER_W)], dst_v)
    plsc.subcore_barrier()

    def body(j, carry):
        pltpu.async_copy(htn_hbm.at[src_v.at[j]], rows_v, sem).wait()
        pltpu.sync_copy(rows_v, shared_agg.at[dst_v.at[j]], add=True)
        return carry

    lax.fori_loop(0, ROWS_PER_W, body, 0)
    plsc.subcore_barrier()
    pltpu.sync_copy(
        shared_agg.at[pl.ds(s * TILE_ROWS, TILE_ROWS)],
        agg_hbm.at[pl.ds(c * N_PAD + s * TILE_ROWS, TILE_ROWS)],
    )


def _tc_prep_body(deg0_ref, deg1_ref, out_ref):
    d = deg0_ref[...][:, 0:1] + deg1_ref[...][:, 0:1] + 1.0
    out_ref[...] = jnp.broadcast_to(lax.rsqrt(d), (BLK, HID))


_tc_prep = pl.pallas_call(
    _tc_prep_body,
    grid=(GRID,),
    in_specs=[
        pl.BlockSpec((BLK, 16), lambda i: (i, 0)),
        pl.BlockSpec((BLK, 16), lambda i: (i + GRID, 0)),
    ],
    out_specs=pl.BlockSpec((BLK, HID), lambda i: (i, 0)),
    out_shape=jax.ShapeDtypeStruct((N_PAD, HID), jnp.float32),
)


def _tc_first_body(x_ref, w_ref, dinv_ref, out_ref):
    out_ref[...] = dinv_ref[...] * jnp.dot(
        x_ref[...], w_ref[...], preferred_element_type=jnp.float32)


_tc_first = pl.pallas_call(
    _tc_first_body,
    grid=(GRID,),
    in_specs=[
        pl.BlockSpec((BLK, IN_LEN), lambda i: (i, 0)),
        pl.BlockSpec((IN_LEN, HID), lambda i: (0, 0)),
        pl.BlockSpec((BLK, HID), lambda i: (i, 0)),
    ],
    out_specs=pl.BlockSpec((BLK, HID), lambda i: (i, 0)),
    out_shape=jax.ShapeDtypeStruct((N_PAD, HID), jnp.float32),
)


def _tc_mid_body(agg0_ref, agg1_ref, htnp_ref, dinv_ref, b_ref, w_ref, out_ref):
    a = agg0_ref[...] + agg1_ref[...] + htnp_ref[...]
    h = jnp.maximum(a * dinv_ref[...] + b_ref[...], 0.0)
    out_ref[...] = dinv_ref[...] * jnp.dot(
        h, w_ref[...], preferred_element_type=jnp.float32)


_tc_mid = pl.pallas_call(
    _tc_mid_body,
    grid=(GRID,),
    in_specs=[
        pl.BlockSpec((BLK, HID), lambda i: (i, 0)),
        pl.BlockSpec((BLK, HID), lambda i: (i + GRID, 0)),
        pl.BlockSpec((BLK, HID), lambda i: (i, 0)),
        pl.BlockSpec((BLK, HID), lambda i: (i, 0)),
        pl.BlockSpec((1, HID), lambda i: (0, 0)),
        pl.BlockSpec((HID, HID), lambda i: (0, 0)),
    ],
    out_specs=pl.BlockSpec((BLK, HID), lambda i: (i, 0)),
    out_shape=jax.ShapeDtypeStruct((N_PAD, HID), jnp.float32),
)


def _tc_last_body(agg0_ref, agg1_ref, htnp_ref, dinv_ref, b_ref, out_ref):
    a = agg0_ref[...] + agg1_ref[...] + htnp_ref[...]
    out_ref[...] = jnp.maximum(a * dinv_ref[...] + b_ref[...], 0.0)


_tc_last = pl.pallas_call(
    _tc_last_body,
    grid=(GRID,),
    in_specs=[
        pl.BlockSpec((BLK, HID), lambda i: (i, 0)),
        pl.BlockSpec((BLK, HID), lambda i: (i + GRID, 0)),
        pl.BlockSpec((BLK, HID), lambda i: (i, 0)),
        pl.BlockSpec((BLK, HID), lambda i: (i, 0)),
        pl.BlockSpec((1, HID), lambda i: (0, 0)),
    ],
    out_specs=pl.BlockSpec((BLK, HID), lambda i: (i, 0)),
    out_shape=jax.ShapeDtypeStruct((N_PAD, HID), jnp.float32),
)


def kernel(X, edge_index, W0, b0, W1, b1, W2, b2, W3, b3, W4, b4):
    pad_e = E_PAD - E
    src_p = jnp.concatenate(
        [edge_index[0], jnp.zeros((pad_e,), jnp.int32)]).reshape(E_ROWS, CHUNK)
    dst_p = jnp.concatenate(
        [edge_index[1], jnp.full((pad_e,), DUMMY, jnp.int32)]).reshape(E_ROWS, CHUNK)
    Xp = jnp.pad(X, ((0, N_PAD - N), (0, 0)))
    zeros16 = jnp.zeros((TILE_ROWS, 16), jnp.float32)
    ones16 = jnp.ones((CHUNK, 16), jnp.float32)
    zeros64 = jnp.zeros((TILE_ROWS, HID), jnp.float32)

    deg = _sc_degree(dst_p, ones16, zeros16)
    dinv = _tc_prep(deg, deg)

    htn = _tc_first(Xp, W0, dinv)
    Ws = [W1, W2, W3, W4]
    bs = [b0.reshape(1, HID), b1.reshape(1, HID), b2.reshape(1, HID),
          b3.reshape(1, HID), b4.reshape(1, HID)]
    for l in range(4):
        agg = _sc_aggregate(htn, src_p, dst_p, zeros64)
        htn = _tc_mid(agg, agg, htn, dinv, bs[l], Ws[l])
    agg = _sc_aggregate(htn, src_p, dst_p, zeros64)
    H = _tc_last(agg, agg, htn, dinv, bs[4])
    return H[:N]


# speedup vs baseline: 24.4081x; 2.5305x over previous
import functools

import jax
import jax.numpy as jnp
from jax import lax
from jax.experimental import pallas as pl
from jax.experimental.pallas import tpu as pltpu
from jax.experimental.pallas import tpu_sc as plsc

N = 10000
N_PAD = 10240
E = 320000
IN_LEN = 128
HID = 64

NC = 2
NS = 16
NW = NC * NS
CHUNK = 128
E_ROWS = 2560
E_PAD = E_ROWS * CHUNK
ROWS_PER_W = E_ROWS // NW
TILE_ROWS = N_PAD // NS
DUMMY = N

BLK = 256
GRID = N_PAD // BLK

_mesh = plsc.VectorSubcoreMesh(core_axis_name="c", subcore_axis_name="s")
_sc_params = pltpu.CompilerParams(use_tc_tiling_on_sc=False)


@functools.partial(
    pl.kernel,
    out_type=jax.ShapeDtypeStruct((NC * N_PAD, 16), jnp.float32),
    mesh=_mesh,
    scratch_types=[
        pltpu.VMEM((ROWS_PER_W, CHUNK), jnp.int32),
        pltpu.VMEM((CHUNK, 16), jnp.float32),
        pltpu.VMEM_SHARED((N_PAD, 16), jnp.float32),
    ],
    compiler_params=_sc_params,
)
def _sc_degree(dst_hbm, ones_hbm, zeros_hbm, deg_hbm, dst_v, ones_v, shared_deg):
    c = lax.axis_index("c")
    s = lax.axis_index("s")
    w = c * NS + s
    pltpu.sync_copy(zeros_hbm, shared_deg.at[pl.ds(s * TILE_ROWS, TILE_ROWS)])
    pltpu.sync_copy(ones_hbm, ones_v)
    pltpu.sync_copy(dst_hbm.at[pl.ds(w * ROWS_PER_W, ROWS_PER_W)], dst_v)
    plsc.subcore_barrier()

    def body(j, carry):
        pltpu.sync_copy(ones_v, shared_deg.at[dst_v.at[j]], add=True)
        return carry

    lax.fori_loop(0, ROWS_PER_W, body, 0)
    plsc.subcore_barrier()
    pltpu.sync_copy(
        shared_deg.at[pl.ds(s * TILE_ROWS, TILE_ROWS)],
        deg_hbm.at[pl.ds(c * N_PAD + s * TILE_ROWS, TILE_ROWS)],
    )


@functools.partial(
    pl.kernel,
    out_type=jax.ShapeDtypeStruct((NC * N_PAD, HID), jnp.float32),
    mesh=_mesh,
    scratch_types=[
        pltpu.VMEM((ROWS_PER_W, CHUNK), jnp.int32),
        pltpu.VMEM((ROWS_PER_W, CHUNK), jnp.int32),
        pltpu.VMEM((CHUNK, HID), jnp.float32),
        pltpu.VMEM((CHUNK, HID), jnp.float32),
        pltpu.VMEM_SHARED((N_PAD, HID), jnp.float32),
        pltpu.SemaphoreType.DMA,
        pltpu.SemaphoreType.DMA,
    ],
    compiler_params=_sc_params,
)
def _sc_aggregate(htn_hbm, src_hbm, dst_hbm, zeros_hbm, agg_hbm,
                  src_v, dst_v, rows0, rows1, shared_agg, sem0, sem1):
    c = lax.axis_index("c")
    s = lax.axis_index("s")
    w = c * NS + s
    pltpu.sync_copy(zeros_hbm, shared_agg.at[pl.ds(s * TILE_ROWS, TILE_ROWS)])
    pltpu.sync_copy(src_hbm.at[pl.ds(w * ROWS_PER_W, ROWS_PER_W)], src_v)
    pltpu.sync_copy(dst_hbm.at[pl.ds(w * ROWS_PER_W, ROWS_PER_W)], dst_v)
    plsc.subcore_barrier()

    pltpu.async_copy(htn_hbm.at[src_v.at[0]], rows0, sem0)

    def body(i, carry):
        j = 2 * i
        pltpu.async_copy(htn_hbm.at[src_v.at[j + 1]], rows1, sem1)
        pltpu.make_async_copy(htn_hbm.at[pl.ds(0, CHUNK)], rows0, sem0).wait()
        pltpu.sync_copy(rows0, shared_agg.at[dst_v.at[j]], add=True)

        @pl.when(j + 2 < ROWS_PER_W)
        def _():
            pltpu.async_copy(htn_hbm.at[src_v.at[j + 2]], rows0, sem0)

        pltpu.make_async_copy(htn_hbm.at[pl.ds(0, CHUNK)], rows1, sem1).wait()
        pltpu.sync_copy(rows1, shared_agg.at[dst_v.at[j + 1]], add=True)
        return carry

    lax.fori_loop(0, ROWS_PER_W // 2, body, 0)
    plsc.subcore_barrier()
    pltpu.sync_copy(
        shared_agg.at[pl.ds(s * TILE_ROWS, TILE_ROWS)],
        agg_hbm.at[pl.ds(c * N_PAD + s * TILE_ROWS, TILE_ROWS)],
    )


def _tc_prep_body(deg0_ref, deg1_ref, out_ref):
    d = deg0_ref[...][:, 0:1] + deg1_ref[...][:, 0:1] + 1.0
    out_ref[...] = jnp.broadcast_to(lax.rsqrt(d), (BLK, HID))


_tc_prep = pl.pallas_call(
    _tc_prep_body,
    grid=(GRID,),
    in_specs=[
        pl.BlockSpec((BLK, 16), lambda i: (i, 0)),
        pl.BlockSpec((BLK, 16), lambda i: (i + GRID, 0)),
    ],
    out_specs=pl.BlockSpec((BLK, HID), lambda i: (i, 0)),
    out_shape=jax.ShapeDtypeStruct((N_PAD, HID), jnp.float32),
)


def _tc_first_body(x_ref, w_ref, dinv_ref, out_ref):
    out_ref[...] = dinv_ref[...] * jnp.dot(
        x_ref[...], w_ref[...], preferred_element_type=jnp.float32)


_tc_first = pl.pallas_call(
    _tc_first_body,
    grid=(GRID,),
    in_specs=[
        pl.BlockSpec((BLK, IN_LEN), lambda i: (i, 0)),
        pl.BlockSpec((IN_LEN, HID), lambda i: (0, 0)),
        pl.BlockSpec((BLK, HID), lambda i: (i, 0)),
    ],
    out_specs=pl.BlockSpec((BLK, HID), lambda i: (i, 0)),
    out_shape=jax.ShapeDtypeStruct((N_PAD, HID), jnp.float32),
)


def _tc_mid_body(agg0_ref, agg1_ref, htnp_ref, dinv_ref, b_ref, w_ref, out_ref):
    a = agg0_ref[...] + agg1_ref[...] + htnp_ref[...]
    h = jnp.maximum(a * dinv_ref[...] + b_ref[...], 0.0)
    out_ref[...] = dinv_ref[...] * jnp.dot(
        h, w_ref[...], preferred_element_type=jnp.float32)


_tc_mid = pl.pallas_call(
    _tc_mid_body,
    grid=(GRID,),
    in_specs=[
        pl.BlockSpec((BLK, HID), lambda i: (i, 0)),
        pl.BlockSpec((BLK, HID), lambda i: (i + GRID, 0)),
        pl.BlockSpec((BLK, HID), lambda i: (i, 0)),
        pl.BlockSpec((BLK, HID), lambda i: (i, 0)),
        pl.BlockSpec((1, HID), lambda i: (0, 0)),
        pl.BlockSpec((HID, HID), lambda i: (0, 0)),
    ],
    out_specs=pl.BlockSpec((BLK, HID), lambda i: (i, 0)),
    out_shape=jax.ShapeDtypeStruct((N_PAD, HID), jnp.float32),
)


def _tc_last_body(agg0_ref, agg1_ref, htnp_ref, dinv_ref, b_ref, out_ref):
    a = agg0_ref[...] + agg1_ref[...] + htnp_ref[...]
    out_ref[...] = jnp.maximum(a * dinv_ref[...] + b_ref[...], 0.0)


_tc_last = pl.pallas_call(
    _tc_last_body,
    grid=(GRID,),
    in_specs=[
        pl.BlockSpec((BLK, HID), lambda i: (i, 0)),
        pl.BlockSpec((BLK, HID), lambda i: (i + GRID, 0)),
        pl.BlockSpec((BLK, HID), lambda i: (i, 0)),
        pl.BlockSpec((BLK, HID), lambda i: (i, 0)),
        pl.BlockSpec((1, HID), lambda i: (0, 0)),
    ],
    out_specs=pl.BlockSpec((BLK, HID), lambda i: (i, 0)),
    out_shape=jax.ShapeDtypeStruct((N_PAD, HID), jnp.float32),
)


def kernel(X, edge_index, W0, b0, W1, b1, W2, b2, W3, b3, W4, b4):
    pad_e = E_PAD - E
    pad_idx = jnp.arange(pad_e, dtype=jnp.int32)
    src_p = jnp.concatenate(
        [edge_index[0], pad_idx % N]).reshape(E_ROWS, CHUNK)
    dst_p = jnp.concatenate(
        [edge_index[1], DUMMY + pad_idx % (N_PAD - N)]).reshape(E_ROWS, CHUNK)
    Xp = jnp.pad(X, ((0, N_PAD - N), (0, 0)))
    zeros16 = jnp.zeros((TILE_ROWS, 16), jnp.float32)
    ones16 = jnp.ones((CHUNK, 16), jnp.float32)
    zeros64 = jnp.zeros((TILE_ROWS, HID), jnp.float32)

    deg = _sc_degree(dst_p, ones16, zeros16)
    dinv = _tc_prep(deg, deg)

    htn = _tc_first(Xp, W0, dinv)
    Ws = [W1, W2, W3, W4]
    bs = [b0.reshape(1, HID), b1.reshape(1, HID), b2.reshape(1, HID),
          b3.reshape(1, HID), b4.reshape(1, HID)]
    for l in range(4):
        agg = _sc_aggregate(htn, src_p, dst_p, zeros64)
        htn = _tc_mid(agg, agg, htn, dinv, bs[l], Ws[l])
    agg = _sc_aggregate(htn, src_p, dst_p, zeros64)
    H = _tc_last(agg, agg, htn, dinv, bs[4])
    return H[:N]


# speedup vs baseline: 29.1998x; 1.1963x over previous
import functools

import jax
import jax.numpy as jnp
from jax import lax
from jax.experimental import pallas as pl
from jax.experimental.pallas import tpu as pltpu
from jax.experimental.pallas import tpu_sc as plsc

N = 10000
N_PAD = 10240
E = 320000
IN_LEN = 128
HID = 64

NC = 2
NS = 16
NW = NC * NS
CHUNK = 128
E_ROWS = 2560
E_PAD = E_ROWS * CHUNK
ROWS_PER_W = E_ROWS // NW
TILE_ROWS = N_PAD // NS
DUMMY = N

BLK = 1024
GRID = N_PAD // BLK

_mesh = plsc.VectorSubcoreMesh(core_axis_name="c", subcore_axis_name="s")
_sc_params = pltpu.CompilerParams(use_tc_tiling_on_sc=False)


@functools.partial(
    pl.kernel,
    out_type=jax.ShapeDtypeStruct((NC * N_PAD, 16), jnp.float32),
    mesh=_mesh,
    scratch_types=[
        pltpu.VMEM((ROWS_PER_W, CHUNK), jnp.int32),
        pltpu.VMEM((CHUNK, 16), jnp.float32),
        pltpu.VMEM_SHARED((N_PAD, 16), jnp.float32),
    ],
    compiler_params=_sc_params,
)
def _sc_degree(dst_hbm, ones_hbm, zeros_hbm, deg_hbm, dst_v, ones_v, shared_deg):
    c = lax.axis_index("c")
    s = lax.axis_index("s")
    w = c * NS + s
    pltpu.sync_copy(zeros_hbm, shared_deg.at[pl.ds(s * TILE_ROWS, TILE_ROWS)])
    pltpu.sync_copy(ones_hbm, ones_v)
    pltpu.sync_copy(dst_hbm.at[pl.ds(w * ROWS_PER_W, ROWS_PER_W)], dst_v)
    plsc.subcore_barrier()

    def body(j, carry):
        pltpu.sync_copy(ones_v, shared_deg.at[dst_v.at[j]], add=True)
        return carry

    lax.fori_loop(0, ROWS_PER_W, body, 0)
    plsc.subcore_barrier()
    pltpu.sync_copy(
        shared_deg.at[pl.ds(s * TILE_ROWS, TILE_ROWS)],
        deg_hbm.at[pl.ds(c * N_PAD + s * TILE_ROWS, TILE_ROWS)],
    )


@functools.partial(
    pl.kernel,
    out_type=jax.ShapeDtypeStruct((NC * N_PAD, HID), jnp.float32),
    mesh=_mesh,
    scratch_types=[
        pltpu.VMEM((ROWS_PER_W, CHUNK), jnp.int32),
        pltpu.VMEM((ROWS_PER_W, CHUNK), jnp.int32),
        pltpu.VMEM((CHUNK, HID), jnp.float32),
        pltpu.VMEM((CHUNK, HID), jnp.float32),
        pltpu.VMEM_SHARED((N_PAD, HID), jnp.float32),
        pltpu.SemaphoreType.DMA,
        pltpu.SemaphoreType.DMA,
    ],
    compiler_params=_sc_params,
)
def _sc_aggregate(htn_hbm, src_hbm, dst_hbm, zeros_hbm, agg_hbm,
                  src_v, dst_v, rows0, rows1, shared_agg, sem0, sem1):
    c = lax.axis_index("c")
    s = lax.axis_index("s")
    w = c * NS + s
    pltpu.sync_copy(zeros_hbm, shared_agg.at[pl.ds(s * TILE_ROWS, TILE_ROWS)])
    pltpu.sync_copy(src_hbm.at[pl.ds(w * ROWS_PER_W, ROWS_PER_W)], src_v)
    pltpu.sync_copy(dst_hbm.at[pl.ds(w * ROWS_PER_W, ROWS_PER_W)], dst_v)
    plsc.subcore_barrier()

    pltpu.async_copy(htn_hbm.at[src_v.at[0]], rows0, sem0)

    def body(i, carry):
        j = 2 * i
        pltpu.async_copy(htn_hbm.at[src_v.at[j + 1]], rows1, sem1)
        pltpu.make_async_copy(htn_hbm.at[pl.ds(0, CHUNK)], rows0, sem0).wait()
        pltpu.sync_copy(rows0, shared_agg.at[dst_v.at[j]], add=True)

        @pl.when(j + 2 < ROWS_PER_W)
        def _():
            pltpu.async_copy(htn_hbm.at[src_v.at[j + 2]], rows0, sem0)

        pltpu.make_async_copy(htn_hbm.at[pl.ds(0, CHUNK)], rows1, sem1).wait()
        pltpu.sync_copy(rows1, shared_agg.at[dst_v.at[j + 1]], add=True)
        return carry

    lax.fori_loop(0, ROWS_PER_W // 2, body, 0)
    plsc.subcore_barrier()
    pltpu.sync_copy(
        shared_agg.at[pl.ds(s * TILE_ROWS, TILE_ROWS)],
        agg_hbm.at[pl.ds(c * N_PAD + s * TILE_ROWS, TILE_ROWS)],
    )


def _tc_first_body(x_ref, w_ref, out_ref):
    out_ref[...] = jnp.dot(
        x_ref[...], w_ref[...], preferred_element_type=jnp.float32)


_tc_first = pl.pallas_call(
    _tc_first_body,
    grid=(GRID,),
    in_specs=[
        pl.BlockSpec((BLK, IN_LEN), lambda i: (i, 0)),
        pl.BlockSpec((IN_LEN, HID), lambda i: (0, 0)),
    ],
    out_specs=pl.BlockSpec((BLK, HID), lambda i: (i, 0)),
    out_shape=jax.ShapeDtypeStruct((N_PAD, HID), jnp.float32),
)


def _tc_scale_body(deg0_ref, deg1_ref, ht_ref, dinv_ref, htn_ref):
    d = deg0_ref[...][:, 0:1] + deg1_ref[...][:, 0:1] + 1.0
    dinv = jnp.broadcast_to(lax.rsqrt(d), (BLK, HID))
    dinv_ref[...] = dinv
    htn_ref[...] = dinv * ht_ref[...]


_tc_scale = pl.pallas_call(
    _tc_scale_body,
    grid=(GRID,),
    in_specs=[
        pl.BlockSpec((BLK, 16), lambda i: (i, 0)),
        pl.BlockSpec((BLK, 16), lambda i: (i + GRID, 0)),
        pl.BlockSpec((BLK, HID), lambda i: (i, 0)),
    ],
    out_specs=[
        pl.BlockSpec((BLK, HID), lambda i: (i, 0)),
        pl.BlockSpec((BLK, HID), lambda i: (i, 0)),
    ],
    out_shape=[
        jax.ShapeDtypeStruct((N_PAD, HID), jnp.float32),
        jax.ShapeDtypeStruct((N_PAD, HID), jnp.float32),
    ],
)


def _tc_mid_body(agg0_ref, agg1_ref, htnp_ref, dinv_ref, b_ref, w_ref, out_ref):
    a = agg0_ref[...] + agg1_ref[...] + htnp_ref[...]
    h = jnp.maximum(a * dinv_ref[...] + b_ref[...], 0.0)
    out_ref[...] = dinv_ref[...] * jnp.dot(
        h, w_ref[...], preferred_element_type=jnp.float32)


_tc_mid = pl.pallas_call(
    _tc_mid_body,
    grid=(GRID,),
    in_specs=[
        pl.BlockSpec((BLK, HID), lambda i: (i, 0)),
        pl.BlockSpec((BLK, HID), lambda i: (i + GRID, 0)),
        pl.BlockSpec((BLK, HID), lambda i: (i, 0)),
        pl.BlockSpec((BLK, HID), lambda i: (i, 0)),
        pl.BlockSpec((1, HID), lambda i: (0, 0)),
        pl.BlockSpec((HID, HID), lambda i: (0, 0)),
    ],
    out_specs=pl.BlockSpec((BLK, HID), lambda i: (i, 0)),
    out_shape=jax.ShapeDtypeStruct((N_PAD, HID), jnp.float32),
)


def _tc_last_body(agg0_ref, agg1_ref, htnp_ref, dinv_ref, b_ref, out_ref):
    a = agg0_ref[...] + agg1_ref[...] + htnp_ref[...]
    out_ref[...] = jnp.maximum(a * dinv_ref[...] + b_ref[...], 0.0)


_tc_last = pl.pallas_call(
    _tc_last_body,
    grid=(GRID,),
    in_specs=[
        pl.BlockSpec((BLK, HID), lambda i: (i, 0)),
        pl.BlockSpec((BLK, HID), lambda i: (i + GRID, 0)),
        pl.BlockSpec((BLK, HID), lambda i: (i, 0)),
        pl.BlockSpec((BLK, HID), lambda i: (i, 0)),
        pl.BlockSpec((1, HID), lambda i: (0, 0)),
    ],
    out_specs=pl.BlockSpec((BLK, HID), lambda i: (i, 0)),
    out_shape=jax.ShapeDtypeStruct((N_PAD, HID), jnp.float32),
)


def kernel(X, edge_index, W0, b0, W1, b1, W2, b2, W3, b3, W4, b4):
    pad_e = E_PAD - E
    pad_idx = jnp.arange(pad_e, dtype=jnp.int32)
    src_p = jnp.concatenate(
        [edge_index[0], pad_idx % N]).reshape(E_ROWS, CHUNK)
    dst_p = jnp.concatenate(
        [edge_index[1], DUMMY + pad_idx % (N_PAD - N)]).reshape(E_ROWS, CHUNK)
    Xp = jnp.pad(X, ((0, N_PAD - N), (0, 0)))
    zeros16 = jnp.zeros((TILE_ROWS, 16), jnp.float32)
    ones16 = jnp.ones((CHUNK, 16), jnp.float32)
    zeros64 = jnp.zeros((TILE_ROWS, HID), jnp.float32)

    deg = _sc_degree(dst_p, ones16, zeros16)
    ht0 = _tc_first(Xp, W0)
    dinv, htn = _tc_scale(deg, deg, ht0)
    Ws = [W1, W2, W3, W4]
    bs = [b0.reshape(1, HID), b1.reshape(1, HID), b2.reshape(1, HID),
          b3.reshape(1, HID), b4.reshape(1, HID)]
    for l in range(4):
        agg = _sc_aggregate(htn, src_p, dst_p, zeros64)
        htn = _tc_mid(agg, agg, htn, dinv, bs[l], Ws[l])
    agg = _sc_aggregate(htn, src_p, dst_p, zeros64)
    H = _tc_last(agg, agg, htn, dinv, bs[4])
    return H[:N]


# speedup vs baseline: 33.9054x; 1.1612x over previous
import functools

import jax
import jax.numpy as jnp
from jax import lax
from jax.experimental import pallas as pl
from jax.experimental.pallas import tpu as pltpu
from jax.experimental.pallas import tpu_sc as plsc

N = 10000
N_PAD = 10240
E = 320000
IN_LEN = 128
HID = 64

NC = 2
NS = 16
NW = NC * NS
CHUNK = 128
E_ROWS = 2560
E_PAD = E_ROWS * CHUNK
ROWS_PER_W = E_ROWS // NW
TILE_ROWS = N_PAD // NS
DUMMY = N

BLK = 1024
GRID = N_PAD // BLK

_mesh = plsc.VectorSubcoreMesh(core_axis_name="c", subcore_axis_name="s")
_sc_params = pltpu.CompilerParams(use_tc_tiling_on_sc=False)


@functools.partial(
    pl.kernel,
    out_type=jax.ShapeDtypeStruct((NC * N_PAD, 16), jnp.float32),
    mesh=_mesh,
    scratch_types=[
        pltpu.VMEM((ROWS_PER_W, CHUNK), jnp.int32),
        pltpu.VMEM((CHUNK, 16), jnp.float32),
        pltpu.VMEM_SHARED((N_PAD, 16), jnp.float32),
    ],
    compiler_params=_sc_params,
)
def _sc_degree(dst_hbm, ones_hbm, zeros_hbm, deg_hbm, dst_v, ones_v, shared_deg):
    c = lax.axis_index("c")
    s = lax.axis_index("s")
    w = c * NS + s
    pltpu.sync_copy(zeros_hbm, shared_deg.at[pl.ds(s * TILE_ROWS, TILE_ROWS)])
    pltpu.sync_copy(ones_hbm, ones_v)
    pltpu.sync_copy(dst_hbm.at[pl.ds(w * ROWS_PER_W, ROWS_PER_W)], dst_v)
    plsc.subcore_barrier()

    def body(j, carry):
        pltpu.sync_copy(ones_v, shared_deg.at[dst_v.at[j]], add=True)
        return carry

    lax.fori_loop(0, ROWS_PER_W, body, 0)
    plsc.subcore_barrier()
    pltpu.sync_copy(
        shared_deg.at[pl.ds(s * TILE_ROWS, TILE_ROWS)],
        deg_hbm.at[pl.ds(c * N_PAD + s * TILE_ROWS, TILE_ROWS)],
    )


@functools.partial(
    pl.kernel,
    out_type=jax.ShapeDtypeStruct((NC * N_PAD, HID), jnp.float32),
    mesh=_mesh,
    scratch_types=[
        pltpu.VMEM((ROWS_PER_W, CHUNK), jnp.int32),
        pltpu.VMEM((ROWS_PER_W, CHUNK), jnp.int32),
        [pltpu.VMEM((CHUNK, HID), jnp.float32)] * 4,
        [pltpu.SemaphoreType.DMA] * 4,
        pltpu.VMEM_SHARED((N_PAD, HID), jnp.float32),
    ],
    compiler_params=_sc_params,
)
def _sc_aggregate(htn_hbm, src_hbm, dst_hbm, zeros_hbm, agg_hbm,
                  src_v, dst_v, rows, sems, shared_agg):
    c = lax.axis_index("c")
    s = lax.axis_index("s")
    w = c * NS + s
    pltpu.sync_copy(zeros_hbm, shared_agg.at[pl.ds(s * TILE_ROWS, TILE_ROWS)])
    pltpu.sync_copy(src_hbm.at[pl.ds(w * ROWS_PER_W, ROWS_PER_W)], src_v)
    pltpu.sync_copy(dst_hbm.at[pl.ds(w * ROWS_PER_W, ROWS_PER_W)], dst_v)
    plsc.subcore_barrier()

    for k in range(3):
        pltpu.async_copy(htn_hbm.at[src_v.at[k]], rows[k], sems[k])

    def body(i, carry):
        j = 4 * i
        for k in range(4):
            @pl.when(j + k + 3 < ROWS_PER_W)
            def _(k=k):
                pltpu.async_copy(
                    htn_hbm.at[src_v.at[j + k + 3]], rows[(k + 3) % 4], sems[(k + 3) % 4])
            pltpu.make_async_copy(
                htn_hbm.at[pl.ds(0, CHUNK)], rows[k], sems[k]).wait()
            pltpu.sync_copy(rows[k], shared_agg.at[dst_v.at[j + k]], add=True)
        return carry

    lax.fori_loop(0, ROWS_PER_W // 4, body, 0)
    plsc.subcore_barrier()
    pltpu.sync_copy(
        shared_agg.at[pl.ds(s * TILE_ROWS, TILE_ROWS)],
        agg_hbm.at[pl.ds(c * N_PAD + s * TILE_ROWS, TILE_ROWS)],
    )


def _tc_first_body(x_ref, w_ref, out_ref):
    out_ref[...] = jnp.dot(
        x_ref[...], w_ref[...], preferred_element_type=jnp.float32)


_tc_first = pl.pallas_call(
    _tc_first_body,
    grid=(GRID,),
    in_specs=[
        pl.BlockSpec((BLK, IN_LEN), lambda i: (i, 0)),
        pl.BlockSpec((IN_LEN, HID), lambda i: (0, 0)),
    ],
    out_specs=pl.BlockSpec((BLK, HID), lambda i: (i, 0)),
    out_shape=jax.ShapeDtypeStruct((N_PAD, HID), jnp.float32),
)


def _tc_scale_body(deg0_ref, deg1_ref, ht_ref, dinv_ref, htn_ref):
    d = deg0_ref[...][:, 0:1] + deg1_ref[...][:, 0:1] + 1.0
    dinv = jnp.broadcast_to(lax.rsqrt(d), (BLK, HID))
    dinv_ref[...] = dinv
    htn_ref[...] = dinv * ht_ref[...]


_tc_scale = pl.pallas_call(
    _tc_scale_body,
    grid=(GRID,),
    in_specs=[
        pl.BlockSpec((BLK, 16), lambda i: (i, 0)),
        pl.BlockSpec((BLK, 16), lambda i: (i + GRID, 0)),
        pl.BlockSpec((BLK, HID), lambda i: (i, 0)),
    ],
    out_specs=[
        pl.BlockSpec((BLK, HID), lambda i: (i, 0)),
        pl.BlockSpec((BLK, HID), lambda i: (i, 0)),
    ],
    out_shape=[
        jax.ShapeDtypeStruct((N_PAD, HID), jnp.float32),
        jax.ShapeDtypeStruct((N_PAD, HID), jnp.float32),
    ],
)


def _tc_mid_body(agg0_ref, agg1_ref, htnp_ref, dinv_ref, b_ref, w_ref, out_ref):
    a = agg0_ref[...] + agg1_ref[...] + htnp_ref[...]
    h = jnp.maximum(a * dinv_ref[...] + b_ref[...], 0.0)
    out_ref[...] = dinv_ref[...] * jnp.dot(
        h, w_ref[...], preferred_element_type=jnp.float32)


_tc_mid = pl.pallas_call(
    _tc_mid_body,
    grid=(GRID,),
    in_specs=[
        pl.BlockSpec((BLK, HID), lambda i: (i, 0)),
        pl.BlockSpec((BLK, HID), lambda i: (i + GRID, 0)),
        pl.BlockSpec((BLK, HID), lambda i: (i, 0)),
        pl.BlockSpec((BLK, HID), lambda i: (i, 0)),
        pl.BlockSpec((1, HID), lambda i: (0, 0)),
        pl.BlockSpec((HID, HID), lambda i: (0, 0)),
    ],
    out_specs=pl.BlockSpec((BLK, HID), lambda i: (i, 0)),
    out_shape=jax.ShapeDtypeStruct((N_PAD, HID), jnp.float32),
)


def _tc_last_body(agg0_ref, agg1_ref, htnp_ref, dinv_ref, b_ref, out_ref):
    a = agg0_ref[...] + agg1_ref[...] + htnp_ref[...]
    out_ref[...] = jnp.maximum(a * dinv_ref[...] + b_ref[...], 0.0)


_tc_last = pl.pallas_call(
    _tc_last_body,
    grid=(GRID,),
    in_specs=[
        pl.BlockSpec((BLK, HID), lambda i: (i, 0)),
        pl.BlockSpec((BLK, HID), lambda i: (i + GRID, 0)),
        pl.BlockSpec((BLK, HID), lambda i: (i, 0)),
        pl.BlockSpec((BLK, HID), lambda i: (i, 0)),
        pl.BlockSpec((1, HID), lambda i: (0, 0)),
    ],
    out_specs=pl.BlockSpec((BLK, HID), lambda i: (i, 0)),
    out_shape=jax.ShapeDtypeStruct((N_PAD, HID), jnp.float32),
)


def kernel(X, edge_index, W0, b0, W1, b1, W2, b2, W3, b3, W4, b4):
    pad_e = E_PAD - E
    pad_idx = jnp.arange(pad_e, dtype=jnp.int32)
    src_p = jnp.concatenate(
        [edge_index[0], pad_idx % N]).reshape(E_ROWS, CHUNK)
    dst_p = jnp.concatenate(
        [edge_index[1], DUMMY + pad_idx % (N_PAD - N)]).reshape(E_ROWS, CHUNK)
    Xp = jnp.pad(X, ((0, N_PAD - N), (0, 0)))
    zeros16 = jnp.zeros((TILE_ROWS, 16), jnp.float32)
    ones16 = jnp.ones((CHUNK, 16), jnp.float32)
    zeros64 = jnp.zeros((TILE_ROWS, HID), jnp.float32)

    deg = _sc_degree(dst_p, ones16, zeros16)
    ht0 = _tc_first(Xp, W0)
    dinv, htn = _tc_scale(deg, deg, ht0)
    Ws = [W1, W2, W3, W4]
    bs = [b0.reshape(1, HID), b1.reshape(1, HID), b2.reshape(1, HID),
          b3.reshape(1, HID), b4.reshape(1, HID)]
    for l in range(4):
        agg = _sc_aggregate(htn, src_p, dst_p, zeros64)
        htn = _tc_mid(agg, agg, htn, dinv, bs[l], Ws[l])
    agg = _sc_aggregate(htn, src_p, dst_p, zeros64)
    H = _tc_last(agg, agg, htn, dinv, bs[4])
    return H[:N]


# speedup vs baseline: 35.3001x; 1.0411x over previous
import functools

import jax
import jax.numpy as jnp
from jax import lax
from jax.experimental import pallas as pl
from jax.experimental.pallas import tpu as pltpu
from jax.experimental.pallas import tpu_sc as plsc

N = 10000
N_PAD = 10240
E = 320000
IN_LEN = 128
HID = 64

NC = 2
NS = 16
NW = NC * NS
CHUNK = 128
E_ROWS = 2560
E_PAD = E_ROWS * CHUNK
ROWS_PER_W = E_ROWS // NW
TILE_ROWS = N_PAD // NS
DUMMY = N

BLK = 2000
GRID = N // BLK

_mesh = plsc.VectorSubcoreMesh(core_axis_name="c", subcore_axis_name="s")
_sc_params = pltpu.CompilerParams(use_tc_tiling_on_sc=False)


@functools.partial(
    pl.kernel,
    out_type=jax.ShapeDtypeStruct((NC * N_PAD, 16), jnp.float32),
    mesh=_mesh,
    scratch_types=[
        pltpu.VMEM((ROWS_PER_W, CHUNK), jnp.int32),
        pltpu.VMEM((CHUNK, 16), jnp.float32),
        pltpu.VMEM_SHARED((N_PAD, 16), jnp.float32),
    ],
    compiler_params=_sc_params,
)
def _sc_degree(dst_hbm, ones_hbm, zeros_hbm, deg_hbm, dst_v, ones_v, shared_deg):
    c = lax.axis_index("c")
    s = lax.axis_index("s")
    w = c * NS + s
    pltpu.sync_copy(zeros_hbm, shared_deg.at[pl.ds(s * TILE_ROWS, TILE_ROWS)])
    pltpu.sync_copy(ones_hbm, ones_v)
    pltpu.sync_copy(dst_hbm.at[pl.ds(w * ROWS_PER_W, ROWS_PER_W)], dst_v)
    plsc.subcore_barrier()

    def body(j, carry):
        pltpu.sync_copy(ones_v, shared_deg.at[dst_v.at[j]], add=True)
        return carry

    lax.fori_loop(0, ROWS_PER_W, body, 0)
    plsc.subcore_barrier()
    pltpu.sync_copy(
        shared_deg.at[pl.ds(s * TILE_ROWS, TILE_ROWS)],
        deg_hbm.at[pl.ds(c * N_PAD + s * TILE_ROWS, TILE_ROWS)],
    )


@functools.partial(
    pl.kernel,
    out_type=jax.ShapeDtypeStruct((NC * N_PAD, HID), jnp.float32),
    mesh=_mesh,
    scratch_types=[
        pltpu.VMEM((ROWS_PER_W, CHUNK), jnp.int32),
        pltpu.VMEM((ROWS_PER_W, CHUNK), jnp.int32),
        [pltpu.VMEM((CHUNK, HID), jnp.float32)] * 4,
        [pltpu.SemaphoreType.DMA] * 4,
        pltpu.VMEM_SHARED((N_PAD, HID), jnp.float32),
    ],
    compiler_params=_sc_params,
)
def _sc_aggregate(htn_hbm, src_hbm, dst_hbm, zeros_hbm, agg_hbm,
                  src_v, dst_v, rows, sems, shared_agg):
    c = lax.axis_index("c")
    s = lax.axis_index("s")
    w = c * NS + s
    pltpu.sync_copy(zeros_hbm, shared_agg.at[pl.ds(s * TILE_ROWS, TILE_ROWS)])
    pltpu.sync_copy(src_hbm.at[pl.ds(w * ROWS_PER_W, ROWS_PER_W)], src_v)
    pltpu.sync_copy(dst_hbm.at[pl.ds(w * ROWS_PER_W, ROWS_PER_W)], dst_v)
    plsc.subcore_barrier()

    for k in range(3):
        pltpu.async_copy(htn_hbm.at[src_v.at[k]], rows[k], sems[k])

    def body(i, carry):
        j = 4 * i
        for k in range(4):
            @pl.when(j + k + 3 < ROWS_PER_W)
            def _(k=k):
                pltpu.async_copy(
                    htn_hbm.at[src_v.at[j + k + 3]], rows[(k + 3) % 4], sems[(k + 3) % 4])
            pltpu.make_async_copy(
                htn_hbm.at[pl.ds(0, CHUNK)], rows[k], sems[k]).wait()
            pltpu.sync_copy(rows[k], shared_agg.at[dst_v.at[j + k]], add=True)
        return carry

    lax.fori_loop(0, ROWS_PER_W // 4, body, 0)
    plsc.subcore_barrier()
    pltpu.sync_copy(
        shared_agg.at[pl.ds(s * TILE_ROWS, TILE_ROWS)],
        agg_hbm.at[pl.ds(c * N_PAD + s * TILE_ROWS, TILE_ROWS)],
    )


def _tc_first_body(x_ref, w_ref, out_ref):
    out_ref[...] = jnp.dot(
        x_ref[...], w_ref[...], preferred_element_type=jnp.float32)


_tc_first = pl.pallas_call(
    _tc_first_body,
    grid=(GRID,),
    in_specs=[
        pl.BlockSpec((BLK, IN_LEN), lambda i: (i, 0)),
        pl.BlockSpec((IN_LEN, HID), lambda i: (0, 0)),
    ],
    out_specs=pl.BlockSpec((BLK, HID), lambda i: (i, 0)),
    out_shape=jax.ShapeDtypeStruct((N, HID), jnp.float32),
)


def _tc_scale_body(deg0_ref, deg1_ref, ht_ref, dinv_ref, htn_ref):
    d = deg0_ref[0][:, 0:1] + deg1_ref[0][:, 0:1] + 1.0
    dinv = jnp.broadcast_to(lax.rsqrt(d), (BLK, HID))
    dinv_ref[...] = dinv
    htn_ref[...] = dinv * ht_ref[...]


_tc_scale = pl.pallas_call(
    _tc_scale_body,
    grid=(GRID,),
    in_specs=[
        pl.BlockSpec((1, BLK, 16), lambda i: (0, i, 0)),
        pl.BlockSpec((1, BLK, 16), lambda i: (1, i, 0)),
        pl.BlockSpec((BLK, HID), lambda i: (i, 0)),
    ],
    out_specs=[
        pl.BlockSpec((BLK, HID), lambda i: (i, 0)),
        pl.BlockSpec((BLK, HID), lambda i: (i, 0)),
    ],
    out_shape=[
        jax.ShapeDtypeStruct((N, HID), jnp.float32),
        jax.ShapeDtypeStruct((N, HID), jnp.float32),
    ],
)


def _tc_mid_body(agg0_ref, agg1_ref, htnp_ref, dinv_ref, b_ref, w_ref, out_ref):
    a = agg0_ref[0] + agg1_ref[0] + htnp_ref[...]
    h = jnp.maximum(a * dinv_ref[...] + b_ref[...], 0.0)
    out_ref[...] = dinv_ref[...] * jnp.dot(
        h, w_ref[...], preferred_element_type=jnp.float32)


_tc_mid = pl.pallas_call(
    _tc_mid_body,
    grid=(GRID,),
    in_specs=[
        pl.BlockSpec((1, BLK, HID), lambda i: (0, i, 0)),
        pl.BlockSpec((1, BLK, HID), lambda i: (1, i, 0)),
        pl.BlockSpec((BLK, HID), lambda i: (i, 0)),
        pl.BlockSpec((BLK, HID), lambda i: (i, 0)),
        pl.BlockSpec((1, HID), lambda i: (0, 0)),
        pl.BlockSpec((HID, HID), lambda i: (0, 0)),
    ],
    out_specs=pl.BlockSpec((BLK, HID), lambda i: (i, 0)),
    out_shape=jax.ShapeDtypeStruct((N, HID), jnp.float32),
)


def _tc_last_body(agg0_ref, agg1_ref, htnp_ref, dinv_ref, b_ref, out_ref):
    a = agg0_ref[0] + agg1_ref[0] + htnp_ref[...]
    out_ref[...] = jnp.maximum(a * dinv_ref[...] + b_ref[...], 0.0)


_tc_last = pl.pallas_call(
    _tc_last_body,
    grid=(GRID,),
    in_specs=[
        pl.BlockSpec((1, BLK, HID), lambda i: (0, i, 0)),
        pl.BlockSpec((1, BLK, HID), lambda i: (1, i, 0)),
        pl.BlockSpec((BLK, HID), lambda i: (i, 0)),
        pl.BlockSpec((BLK, HID), lambda i: (i, 0)),
        pl.BlockSpec((1, HID), lambda i: (0, 0)),
    ],
    out_specs=pl.BlockSpec((BLK, HID), lambda i: (i, 0)),
    out_shape=jax.ShapeDtypeStruct((N, HID), jnp.float32),
)


def kernel(X, edge_index, W0, b0, W1, b1, W2, b2, W3, b3, W4, b4):
    pad_e = E_PAD - E
    pad_idx = jnp.arange(pad_e, dtype=jnp.int32)
    src_p = jnp.concatenate(
        [edge_index[0], pad_idx % N]).reshape(E_ROWS, CHUNK)
    dst_p = jnp.concatenate(
        [edge_index[1], DUMMY + pad_idx % (N_PAD - N)]).reshape(E_ROWS, CHUNK)
    zeros16 = jnp.zeros((TILE_ROWS, 16), jnp.float32)
    ones16 = jnp.ones((CHUNK, 16), jnp.float32)
    zeros64 = jnp.zeros((TILE_ROWS, HID), jnp.float32)

    deg = _sc_degree(dst_p, ones16, zeros16).reshape(NC, N_PAD, 16)
    ht0 = _tc_first(X, W0)
    dinv, htn = _tc_scale(deg, deg, ht0)
    Ws = [W1, W2, W3, W4]
    bs = [b0.reshape(1, HID), b1.reshape(1, HID), b2.reshape(1, HID),
          b3.reshape(1, HID), b4.reshape(1, HID)]
    for l in range(4):
        agg = _sc_aggregate(htn, src_p, dst_p, zeros64).reshape(NC, N_PAD, HID)
        htn = _tc_mid(agg, agg, htn, dinv, bs[l], Ws[l])
    agg = _sc_aggregate(htn, src_p, dst_p, zeros64).reshape(NC, N_PAD, HID)
    return _tc_last(agg, agg, htn, dinv, bs[4])


# speedup vs baseline: 35.3043x; 1.0001x over previous
import functools

import jax
import jax.numpy as jnp
from jax import lax
from jax.experimental import pallas as pl
from jax.experimental.pallas import tpu as pltpu
from jax.experimental.pallas import tpu_sc as plsc

N = 10000
N_PAD = 10240
E = 320000
IN_LEN = 128
HID = 64

NC = 2
NS = 16
NW = NC * NS
CHUNK = 128
E_ROWS = 2560
E_PAD = E_ROWS * CHUNK
ROWS_PER_W = E_ROWS // NW
TILE_ROWS = N_PAD // NS
DUMMY = N

BLK = 2000
GRID = N // BLK

_mesh = plsc.VectorSubcoreMesh(core_axis_name="c", subcore_axis_name="s")
_sc_params = pltpu.CompilerParams(use_tc_tiling_on_sc=False)


@functools.partial(
    pl.kernel,
    out_type=jax.ShapeDtypeStruct((NC * N_PAD, 16), jnp.float32),
    mesh=_mesh,
    scratch_types=[
        pltpu.VMEM((ROWS_PER_W, CHUNK), jnp.int32),
        pltpu.VMEM((CHUNK, 16), jnp.float32),
        pltpu.VMEM_SHARED((N_PAD, 16), jnp.float32),
    ],
    compiler_params=_sc_params,
)
def _sc_degree(dst_hbm, ones_hbm, zeros_hbm, deg_hbm, dst_v, ones_v, shared_deg):
    c = lax.axis_index("c")
    s = lax.axis_index("s")
    w = c * NS + s
    pltpu.sync_copy(zeros_hbm, shared_deg.at[pl.ds(s * TILE_ROWS, TILE_ROWS)])
    pltpu.sync_copy(ones_hbm, ones_v)
    pltpu.sync_copy(dst_hbm.at[pl.ds(w * ROWS_PER_W, ROWS_PER_W)], dst_v)
    plsc.subcore_barrier()

    def body(j, carry):
        pltpu.sync_copy(ones_v, shared_deg.at[dst_v.at[j]], add=True)
        return carry

    lax.fori_loop(0, ROWS_PER_W, body, 0)
    plsc.subcore_barrier()
    pltpu.sync_copy(
        shared_deg.at[pl.ds(s * TILE_ROWS, TILE_ROWS)],
        deg_hbm.at[pl.ds(c * N_PAD + s * TILE_ROWS, TILE_ROWS)],
    )


@functools.partial(
    pl.kernel,
    out_type=jax.ShapeDtypeStruct((NC * N_PAD, HID), jnp.float32),
    mesh=_mesh,
    scratch_types=[
        pltpu.VMEM((ROWS_PER_W, CHUNK), jnp.int32),
        pltpu.VMEM((ROWS_PER_W, CHUNK), jnp.int32),
        [pltpu.VMEM((CHUNK, HID), jnp.float32)] * 4,
        [pltpu.SemaphoreType.DMA] * 4,
        [pltpu.SemaphoreType.DMA] * 4,
        pltpu.VMEM_SHARED((N_PAD, HID), jnp.float32),
    ],
    compiler_params=_sc_params,
)
def _sc_aggregate(htn_hbm, src_hbm, dst_hbm, zeros_hbm, agg_hbm,
                  src_v, dst_v, rows, sems, ssems, shared_agg):
    c = lax.axis_index("c")
    s = lax.axis_index("s")
    w = c * NS + s
    pltpu.sync_copy(zeros_hbm, shared_agg.at[pl.ds(s * TILE_ROWS, TILE_ROWS)])
    pltpu.sync_copy(src_hbm.at[pl.ds(w * ROWS_PER_W, ROWS_PER_W)], src_v)
    pltpu.sync_copy(dst_hbm.at[pl.ds(w * ROWS_PER_W, ROWS_PER_W)], dst_v)
    plsc.subcore_barrier()

    for k in range(3):
        pltpu.async_copy(htn_hbm.at[src_v.at[k]], rows[k], sems[k])

    def body(i, carry):
        j = 4 * i
        for k in range(4):
            q = j + k + 3
            b = (k + 3) % 4

            @pl.when((q >= 4) & (q < ROWS_PER_W))
            def _(b=b):
                pltpu.make_async_copy(
                    rows[b], shared_agg.at[pl.ds(0, CHUNK)], ssems[b]).wait()

            @pl.when(q < ROWS_PER_W)
            def _(q=q, b=b):
                pltpu.async_copy(htn_hbm.at[src_v.at[q]], rows[b], sems[b])

            pltpu.make_async_copy(
                htn_hbm.at[pl.ds(0, CHUNK)], rows[k], sems[k]).wait()
            pltpu.async_copy(rows[k], shared_agg.at[dst_v.at[j + k]],
                             ssems[k], add=True)
        return carry

    lax.fori_loop(0, ROWS_PER_W // 4, body, 0)
    for k in range(4):
        pltpu.make_async_copy(rows[k], shared_agg.at[pl.ds(0, CHUNK)], ssems[k]).wait()
    plsc.subcore_barrier()
    pltpu.sync_copy(
        shared_agg.at[pl.ds(s * TILE_ROWS, TILE_ROWS)],
        agg_hbm.at[pl.ds(c * N_PAD + s * TILE_ROWS, TILE_ROWS)],
    )


def _tc_first_body(x_ref, w_ref, out_ref):
    out_ref[...] = jnp.dot(
        x_ref[...], w_ref[...], preferred_element_type=jnp.float32)


_tc_first = pl.pallas_call(
    _tc_first_body,
    grid=(GRID,),
    in_specs=[
        pl.BlockSpec((BLK, IN_LEN), lambda i: (i, 0)),
        pl.BlockSpec((IN_LEN, HID), lambda i: (0, 0)),
    ],
    out_specs=pl.BlockSpec((BLK, HID), lambda i: (i, 0)),
    out_shape=jax.ShapeDtypeStruct((N, HID), jnp.float32),
)


def _tc_scale_body(deg0_ref, deg1_ref, ht_ref, dinv_ref, htn_ref):
    d = deg0_ref[0][:, 0:1] + deg1_ref[0][:, 0:1] + 1.0
    dinv = jnp.broadcast_to(lax.rsqrt(d), (BLK, HID))
    dinv_ref[...] = dinv
    htn_ref[...] = dinv * ht_ref[...]


_tc_scale = pl.pallas_call(
    _tc_scale_body,
    grid=(GRID,),
    in_specs=[
        pl.BlockSpec((1, BLK, 16), lambda i: (0, i, 0)),
        pl.BlockSpec((1, BLK, 16), lambda i: (1, i, 0)),
        pl.BlockSpec((BLK, HID), lambda i: (i, 0)),
    ],
    out_specs=[
        pl.BlockSpec((BLK, HID), lambda i: (i, 0)),
        pl.BlockSpec((BLK, HID), lambda i: (i, 0)),
    ],
    out_shape=[
        jax.ShapeDtypeStruct((N, HID), jnp.float32),
        jax.ShapeDtypeStruct((N, HID), jnp.float32),
    ],
)


def _tc_mid_body(agg0_ref, agg1_ref, htnp_ref, dinv_ref, b_ref, w_ref, out_ref):
    a = agg0_ref[0] + agg1_ref[0] + htnp_ref[...]
    h = jnp.maximum(a * dinv_ref[...] + b_ref[...], 0.0)
    out_ref[...] = dinv_ref[...] * jnp.dot(
        h, w_ref[...], preferred_element_type=jnp.float32)


_tc_mid = pl.pallas_call(
    _tc_mid_body,
    grid=(GRID,),
    in_specs=[
        pl.BlockSpec((1, BLK, HID), lambda i: (0, i, 0)),
        pl.BlockSpec((1, BLK, HID), lambda i: (1, i, 0)),
        pl.BlockSpec((BLK, HID), lambda i: (i, 0)),
        pl.BlockSpec((BLK, HID), lambda i: (i, 0)),
        pl.BlockSpec((1, HID), lambda i: (0, 0)),
        pl.BlockSpec((HID, HID), lambda i: (0, 0)),
    ],
    out_specs=pl.BlockSpec((BLK, HID), lambda i: (i, 0)),
    out_shape=jax.ShapeDtypeStruct((N, HID), jnp.float32),
)


def _tc_last_body(agg0_ref, agg1_ref, htnp_ref, dinv_ref, b_ref, out_ref):
    a = agg0_ref[0] + agg1_ref[0] + htnp_ref[...]
    out_ref[...] = jnp.maximum(a * dinv_ref[...] + b_ref[...], 0.0)


_tc_last = pl.pallas_call(
    _tc_last_body,
    grid=(GRID,),
    in_specs=[
        pl.BlockSpec((1, BLK, HID), lambda i: (0, i, 0)),
        pl.BlockSpec((1, BLK, HID), lambda i: (1, i, 0)),
        pl.BlockSpec((BLK, HID), lambda i: (i, 0)),
        pl.BlockSpec((BLK, HID), lambda i: (i, 0)),
        pl.BlockSpec((1, HID), lambda i: (0, 0)),
    ],
    out_specs=pl.BlockSpec((BLK, HID), lambda i: (i, 0)),
    out_shape=jax.ShapeDtypeStruct((N, HID), jnp.float32),
)


def kernel(X, edge_index, W0, b0, W1, b1, W2, b2, W3, b3, W4, b4):
    pad_e = E_PAD - E
    pad_idx = jnp.arange(pad_e, dtype=jnp.int32)
    src_p = jnp.concatenate(
        [edge_index[0], pad_idx % N]).reshape(E_ROWS, CHUNK)
    dst_p = jnp.concatenate(
        [edge_index[1], DUMMY + pad_idx % (N_PAD - N)]).reshape(E_ROWS, CHUNK)
    zeros16 = jnp.zeros((TILE_ROWS, 16), jnp.float32)
    ones16 = jnp.ones((CHUNK, 16), jnp.float32)
    zeros64 = jnp.zeros((TILE_ROWS, HID), jnp.float32)

    deg = _sc_degree(dst_p, ones16, zeros16).reshape(NC, N_PAD, 16)
    ht0 = _tc_first(X, W0)
    dinv, htn = _tc_scale(deg, deg, ht0)
    Ws = [W1, W2, W3, W4]
    bs = [b0.reshape(1, HID), b1.reshape(1, HID), b2.reshape(1, HID),
          b3.reshape(1, HID), b4.reshape(1, HID)]
    for l in range(4):
        agg = _sc_aggregate(htn, src_p, dst_p, zeros64).reshape(NC, N_PAD, HID)
        htn = _tc_mid(agg, agg, htn, dinv, bs[l], Ws[l])
    agg = _sc_aggregate(htn, src_p, dst_p, zeros64).reshape(NC, N_PAD, HID)
    return _tc_last(agg, agg, htn, dinv, bs[4])


# speedup vs baseline: 39.4993x; 1.1188x over previous
import functools

import jax
import jax.numpy as jnp
from jax import lax
from jax.experimental import pallas as pl
from jax.experimental.pallas import tpu as pltpu
from jax.experimental.pallas import tpu_sc as plsc

N = 10000
N_PAD = 10240
E = 320000
IN_LEN = 128
HID = 64

NC = 2
NS = 16
NW = NC * NS
CHUNK = 128
E_ROWS = 2560
E_PAD = E_ROWS * CHUNK
ROWS_PER_W = E_ROWS // NW
TILE_ROWS = N_PAD // NS
DUMMY = N

BLK = 2000
GRID = N // BLK

_mesh = plsc.VectorSubcoreMesh(core_axis_name="c", subcore_axis_name="s")
_sc_params = pltpu.CompilerParams(use_tc_tiling_on_sc=False)


@functools.partial(
    pl.kernel,
    out_type=jax.ShapeDtypeStruct((NC * N_PAD, 16), jnp.float32),
    mesh=_mesh,
    scratch_types=[
        pltpu.VMEM((ROWS_PER_W, CHUNK), jnp.int32),
        pltpu.VMEM((CHUNK, 16), jnp.float32),
        pltpu.VMEM_SHARED((N_PAD, 16), jnp.float32),
    ],
    compiler_params=_sc_params,
)
def _sc_degree(dst_hbm, ones_hbm, zeros_hbm, deg_hbm, dst_v, ones_v, shared_deg):
    c = lax.axis_index("c")
    s = lax.axis_index("s")
    w = c * NS + s
    pltpu.sync_copy(zeros_hbm, shared_deg.at[pl.ds(s * TILE_ROWS, TILE_ROWS)])
    pltpu.sync_copy(ones_hbm, ones_v)
    pltpu.sync_copy(dst_hbm.at[pl.ds(w * ROWS_PER_W, ROWS_PER_W)], dst_v)
    plsc.subcore_barrier()

    def body(j, carry):
        pltpu.sync_copy(ones_v, shared_deg.at[dst_v.at[j]], add=True)
        return carry

    lax.fori_loop(0, ROWS_PER_W, body, 0)
    plsc.subcore_barrier()
    pltpu.sync_copy(
        shared_deg.at[pl.ds(s * TILE_ROWS, TILE_ROWS)],
        deg_hbm.at[pl.ds(c * N_PAD + s * TILE_ROWS, TILE_ROWS)],
    )


@functools.partial(
    pl.kernel,
    out_type=jax.ShapeDtypeStruct((N_PAD, NC * HID), jnp.float32),
    mesh=_mesh,
    scratch_types=[
        pltpu.VMEM((ROWS_PER_W, CHUNK), jnp.int32),
        pltpu.VMEM((ROWS_PER_W, CHUNK), jnp.int32),
        [pltpu.VMEM((CHUNK, HID), jnp.float32)] * 4,
        [pltpu.SemaphoreType.DMA] * 4,
        [pltpu.SemaphoreType.DMA] * 4,
        pltpu.VMEM_SHARED((N_PAD, HID), jnp.float32),
    ],
    compiler_params=_sc_params,
)
def _sc_aggregate(htn_hbm, src_hbm, dst_hbm, zeros_hbm, agg_hbm,
                  src_v, dst_v, rows, sems, ssems, shared_agg):
    c = lax.axis_index("c")
    s = lax.axis_index("s")
    w = c * NS + s
    pltpu.sync_copy(zeros_hbm, shared_agg.at[pl.ds(s * TILE_ROWS, TILE_ROWS)])
    pltpu.sync_copy(src_hbm.at[pl.ds(w * ROWS_PER_W, ROWS_PER_W)], src_v)
    pltpu.sync_copy(dst_hbm.at[pl.ds(w * ROWS_PER_W, ROWS_PER_W)], dst_v)
    plsc.subcore_barrier()

    for k in range(3):
        pltpu.async_copy(htn_hbm.at[src_v.at[k]], rows[k], sems[k])

    def body(i, carry):
        j = 4 * i
        for k in range(4):
            q = j + k + 3
            b = (k + 3) % 4

            @pl.when((q >= 4) & (q < ROWS_PER_W))
            def _(b=b):
                pltpu.make_async_copy(
                    rows[b], shared_agg.at[pl.ds(0, CHUNK)], ssems[b]).wait()

            @pl.when(q < ROWS_PER_W)
            def _(q=q, b=b):
                pltpu.async_copy(htn_hbm.at[src_v.at[q]], rows[b], sems[b])

            pltpu.make_async_copy(
                htn_hbm.at[pl.ds(0, CHUNK)], rows[k], sems[k]).wait()
            pltpu.async_copy(rows[k], shared_agg.at[dst_v.at[j + k]],
                             ssems[k], add=True)
        return carry

    lax.fori_loop(0, ROWS_PER_W // 4, body, 0)
    for k in range(4):
        pltpu.make_async_copy(rows[k], shared_agg.at[pl.ds(0, CHUNK)], ssems[k]).wait()
    plsc.subcore_barrier()
    pltpu.sync_copy(
        shared_agg.at[pl.ds(s * TILE_ROWS, TILE_ROWS)],
        agg_hbm.at[pl.ds(s * TILE_ROWS, TILE_ROWS), pl.ds(c * HID, HID)],
    )


def _tc_first_body(x_ref, w_ref, out_ref):
    out_ref[...] = jnp.dot(
        x_ref[...], w_ref[...], preferred_element_type=jnp.float32)


_tc_first = pl.pallas_call(
    _tc_first_body,
    grid=(GRID,),
    in_specs=[
        pl.BlockSpec((BLK, IN_LEN), lambda i: (i, 0)),
        pl.BlockSpec((IN_LEN, HID), lambda i: (0, 0)),
    ],
    out_specs=pl.BlockSpec((BLK, HID), lambda i: (i, 0)),
    out_shape=jax.ShapeDtypeStruct((N, HID), jnp.float32),
)


def _tc_scale_body(deg0_ref, deg1_ref, ht_ref, dinv_ref, htn_ref):
    d = deg0_ref[0][:, 0:1] + deg1_ref[0][:, 0:1] + 1.0
    dinv = jnp.broadcast_to(lax.rsqrt(d), (BLK, HID))
    dinv_ref[...] = dinv
    htn_ref[...] = dinv * ht_ref[...]


_tc_scale = pl.pallas_call(
    _tc_scale_body,
    grid=(GRID,),
    in_specs=[
        pl.BlockSpec((1, BLK, 16), lambda i: (0, i, 0)),
        pl.BlockSpec((1, BLK, 16), lambda i: (1, i, 0)),
        pl.BlockSpec((BLK, HID), lambda i: (i, 0)),
    ],
    out_specs=[
        pl.BlockSpec((BLK, HID), lambda i: (i, 0)),
        pl.BlockSpec((BLK, HID), lambda i: (i, 0)),
    ],
    out_shape=[
        jax.ShapeDtypeStruct((N, HID), jnp.float32),
        jax.ShapeDtypeStruct((N, HID), jnp.float32),
    ],
)


def _tc_mid_body(agg_ref, htnp_ref, dinv_ref, b_ref, w_ref, out_ref):
    a = agg_ref[:, :HID] + agg_ref[:, HID:] + htnp_ref[...]
    h = jnp.maximum(a * dinv_ref[...] + b_ref[...], 0.0)
    out_ref[...] = dinv_ref[...] * jnp.dot(
        h, w_ref[...], preferred_element_type=jnp.float32)


_tc_mid = pl.pallas_call(
    _tc_mid_body,
    grid=(GRID,),
    in_specs=[
        pl.BlockSpec((BLK, NC * HID), lambda i: (i, 0)),
        pl.BlockSpec((BLK, HID), lambda i: (i, 0)),
        pl.BlockSpec((BLK, HID), lambda i: (i, 0)),
        pl.BlockSpec((1, HID), lambda i: (0, 0)),
        pl.BlockSpec((HID, HID), lambda i: (0, 0)),
    ],
    out_specs=pl.BlockSpec((BLK, HID), lambda i: (i, 0)),
    out_shape=jax.ShapeDtypeStruct((N, HID), jnp.float32),
)


def _tc_last_body(agg_ref, htnp_ref, dinv_ref, b_ref, out_ref):
    a = agg_ref[:, :HID] + agg_ref[:, HID:] + htnp_ref[...]
    out_ref[...] = jnp.maximum(a * dinv_ref[...] + b_ref[...], 0.0)


_tc_last = pl.pallas_call(
    _tc_last_body,
    grid=(GRID,),
    in_specs=[
        pl.BlockSpec((BLK, NC * HID), lambda i: (i, 0)),
        pl.BlockSpec((BLK, HID), lambda i: (i, 0)),
        pl.BlockSpec((BLK, HID), lambda i: (i, 0)),
        pl.BlockSpec((1, HID), lambda i: (0, 0)),
    ],
    out_specs=pl.BlockSpec((BLK, HID), lambda i: (i, 0)),
    out_shape=jax.ShapeDtypeStruct((N, HID), jnp.float32),
)


def kernel(X, edge_index, W0, b0, W1, b1, W2, b2, W3, b3, W4, b4):
    pad_e = E_PAD - E
    pad_idx = jnp.arange(pad_e, dtype=jnp.int32)
    src_p = jnp.concatenate(
        [edge_index[0], pad_idx % N]).reshape(E_ROWS, CHUNK)
    dst_p = jnp.concatenate(
        [edge_index[1], DUMMY + pad_idx % (N_PAD - N)]).reshape(E_ROWS, CHUNK)
    zeros16 = jnp.zeros((TILE_ROWS, 16), jnp.float32)
    ones16 = jnp.ones((CHUNK, 16), jnp.float32)
    zeros64 = jnp.zeros((TILE_ROWS, HID), jnp.float32)

    deg = _sc_degree(dst_p, ones16, zeros16).reshape(NC, N_PAD, 16)
    ht0 = _tc_first(X, W0)
    dinv, htn = _tc_scale(deg, deg, ht0)
    Ws = [W1, W2, W3, W4]
    bs = [b0.reshape(1, HID), b1.reshape(1, HID), b2.reshape(1, HID),
          b3.reshape(1, HID), b4.reshape(1, HID)]
    for l in range(4):
        agg = _sc_aggregate(htn, src_p, dst_p, zeros64)
        htn = _tc_mid(agg, htn, dinv, bs[l], Ws[l])
    agg = _sc_aggregate(htn, src_p, dst_p, zeros64)
    return _tc_last(agg, htn, dinv, bs[4])


# speedup vs baseline: 41.2035x; 1.0431x over previous
import functools

import jax
import jax.numpy as jnp
from jax import lax
from jax.experimental import pallas as pl
from jax.experimental.pallas import tpu as pltpu
from jax.experimental.pallas import tpu_sc as plsc

N = 10000
N_PAD = 10240
E = 320000
IN_LEN = 128
HID = 64

NC = 2
NS = 16
NW = NC * NS
CHUNK = 128
E_ROWS = 2560
E_REAL_ROWS = E // CHUNK
ROWS_PER_W = E_ROWS // NW
TAIL_OFF = (NW - 1) * ROWS_PER_W
REAL_TAIL = E_REAL_ROWS - TAIL_OFF
PAD_ROWS = E_ROWS - E_REAL_ROWS
TILE_ROWS = N_PAD // NS
DUMMY = N

BLK = 2000
GRID = N // BLK

_mesh = plsc.VectorSubcoreMesh(core_axis_name="c", subcore_axis_name="s")
_sc_params = pltpu.CompilerParams(use_tc_tiling_on_sc=False)


def _load_edges(ei_hbm, pad_hbm, row, w, idx_v):
    @pl.when(w < NW - 1)
    def _():
        pltpu.sync_copy(ei_hbm.at[row, pl.ds(w * ROWS_PER_W, ROWS_PER_W)], idx_v)

    @pl.when(w == NW - 1)
    def _():
        pltpu.sync_copy(ei_hbm.at[row, pl.ds(TAIL_OFF, REAL_TAIL)],
                        idx_v.at[pl.ds(0, REAL_TAIL)])
        pltpu.sync_copy(pad_hbm.at[row], idx_v.at[pl.ds(REAL_TAIL, PAD_ROWS)])


@functools.partial(
    pl.kernel,
    out_type=jax.ShapeDtypeStruct((NC * N_PAD, 16), jnp.float32),
    mesh=_mesh,
    scratch_types=[
        pltpu.VMEM((ROWS_PER_W, CHUNK), jnp.int32),
        pltpu.VMEM((CHUNK, 16), jnp.float32),
        [pltpu.SemaphoreType.DMA] * 4,
        pltpu.VMEM_SHARED((N_PAD, 16), jnp.float32),
    ],
    compiler_params=_sc_params,
)
def _sc_degree(ei_hbm, pad_hbm, ones_hbm, zeros_hbm, deg_hbm,
               dst_v, ones_v, ssems, shared_deg):
    c = lax.axis_index("c")
    s = lax.axis_index("s")
    w = c * NS + s
    pltpu.sync_copy(zeros_hbm, shared_deg.at[pl.ds(s * TILE_ROWS, TILE_ROWS)])
    pltpu.sync_copy(ones_hbm, ones_v)
    _load_edges(ei_hbm, pad_hbm, 1, w, dst_v)
    plsc.subcore_barrier()

    def body(i, carry):
        for k in range(4):
            @pl.when(i >= 1)
            def _(k=k):
                pltpu.make_async_copy(
                    ones_v, shared_deg.at[pl.ds(0, CHUNK)], ssems[k]).wait()
            pltpu.async_copy(ones_v, shared_deg.at[dst_v.at[4 * i + k]],
                             ssems[k], add=True)
        return carry

    lax.fori_loop(0, ROWS_PER_W // 4, body, 0)
    for k in range(4):
        pltpu.make_async_copy(ones_v, shared_deg.at[pl.ds(0, CHUNK)], ssems[k]).wait()
    plsc.subcore_barrier()
    pltpu.sync_copy(
        shared_deg.at[pl.ds(s * TILE_ROWS, TILE_ROWS)],
        deg_hbm.at[pl.ds(c * N_PAD + s * TILE_ROWS, TILE_ROWS)],
    )


@functools.partial(
    pl.kernel,
    out_type=jax.ShapeDtypeStruct((N_PAD, NC * HID), jnp.float32),
    mesh=_mesh,
    scratch_types=[
        pltpu.VMEM((ROWS_PER_W, CHUNK), jnp.int32),
        pltpu.VMEM((ROWS_PER_W, CHUNK), jnp.int32),
        [pltpu.VMEM((CHUNK, HID), jnp.float32)] * 4,
        [pltpu.SemaphoreType.DMA] * 4,
        [pltpu.SemaphoreType.DMA] * 4,
        pltpu.VMEM_SHARED((N_PAD, HID), jnp.float32),
    ],
    compiler_params=_sc_params,
)
def _sc_aggregate(htn_hbm, ei_hbm, pad_hbm, zeros_hbm, agg_hbm,
                  src_v, dst_v, rows, sems, ssems, shared_agg):
    c = lax.axis_index("c")
    s = lax.axis_index("s")
    w = c * NS + s
    _load_edges(ei_hbm, pad_hbm, 0, w, src_v)
    _load_edges(ei_hbm, pad_hbm, 1, w, dst_v)

    for k in range(3):
        pltpu.async_copy(htn_hbm.at[src_v.at[k]], rows[k], sems[k])
    pltpu.sync_copy(zeros_hbm, shared_agg.at[pl.ds(s * TILE_ROWS, TILE_ROWS)])
    plsc.subcore_barrier()

    def body(i, carry):
        j = 4 * i
        for k in range(4):
            q = j + k + 3
            b = (k + 3) % 4

            @pl.when((q >= 4) & (q < ROWS_PER_W))
            def _(b=b):
                pltpu.make_async_copy(
                    rows[b], shared_agg.at[pl.ds(0, CHUNK)], ssems[b]).wait()

            @pl.when(q < ROWS_PER_W)
            def _(q=q, b=b):
                pltpu.async_copy(htn_hbm.at[src_v.at[q]], rows[b], sems[b])

            pltpu.make_async_copy(
                htn_hbm.at[pl.ds(0, CHUNK)], rows[k], sems[k]).wait()
            pltpu.async_copy(rows[k], shared_agg.at[dst_v.at[j + k]],
                             ssems[k], add=True)
        return carry

    lax.fori_loop(0, ROWS_PER_W // 4, body, 0)
    for k in range(4):
        pltpu.make_async_copy(rows[k], shared_agg.at[pl.ds(0, CHUNK)], ssems[k]).wait()
    plsc.subcore_barrier()
    pltpu.sync_copy(
        shared_agg.at[pl.ds(s * TILE_ROWS, TILE_ROWS)],
        agg_hbm.at[pl.ds(s * TILE_ROWS, TILE_ROWS), pl.ds(c * HID, HID)],
    )


def _tc_first_body(x_ref, w_ref, out_ref):
    out_ref[...] = jnp.dot(
        x_ref[...], w_ref[...], preferred_element_type=jnp.float32)


_tc_first = pl.pallas_call(
    _tc_first_body,
    grid=(GRID,),
    in_specs=[
        pl.BlockSpec((BLK, IN_LEN), lambda i: (i, 0)),
        pl.BlockSpec((IN_LEN, HID), lambda i: (0, 0)),
    ],
    out_specs=pl.BlockSpec((BLK, HID), lambda i: (i, 0)),
    out_shape=jax.ShapeDtypeStruct((N, HID), jnp.float32),
)


def _tc_scale_body(deg0_ref, deg1_ref, ht_ref, dinv_ref, htn_ref):
    d = deg0_ref[0][:, 0:1] + deg1_ref[0][:, 0:1] + 1.0
    dinv = jnp.broadcast_to(lax.rsqrt(d), (BLK, HID))
    dinv_ref[...] = dinv
    htn_ref[...] = dinv * ht_ref[...]


_tc_scale = pl.pallas_call(
    _tc_scale_body,
    grid=(GRID,),
    in_specs=[
        pl.BlockSpec((1, BLK, 16), lambda i: (0, i, 0)),
        pl.BlockSpec((1, BLK, 16), lambda i: (1, i, 0)),
        pl.BlockSpec((BLK, HID), lambda i: (i, 0)),
    ],
    out_specs=[
        pl.BlockSpec((BLK, HID), lambda i: (i, 0)),
        pl.BlockSpec((BLK, HID), lambda i: (i, 0)),
    ],
    out_shape=[
        jax.ShapeDtypeStruct((N, HID), jnp.float32),
        jax.ShapeDtypeStruct((N, HID), jnp.float32),
    ],
)


def _tc_mid_body(agg_ref, htnp_ref, dinv_ref, b_ref, w_ref, out_ref):
    a = agg_ref[:, :HID] + agg_ref[:, HID:] + htnp_ref[...]
    h = jnp.maximum(a * dinv_ref[...] + b_ref[...], 0.0)
    out_ref[...] = dinv_ref[...] * jnp.dot(
        h, w_ref[...], preferred_element_type=jnp.float32)


_tc_mid = pl.pallas_call(
    _tc_mid_body,
    grid=(GRID,),
    in_specs=[
        pl.BlockSpec((BLK, NC * HID), lambda i: (i, 0)),
        pl.BlockSpec((BLK, HID), lambda i: (i, 0)),
        pl.BlockSpec((BLK, HID), lambda i: (i, 0)),
        pl.BlockSpec((1, HID), lambda i: (0, 0)),
        pl.BlockSpec((HID, HID), lambda i: (0, 0)),
    ],
    out_specs=pl.BlockSpec((BLK, HID), lambda i: (i, 0)),
    out_shape=jax.ShapeDtypeStruct((N, HID), jnp.float32),
)


def _tc_last_body(agg_ref, htnp_ref, dinv_ref, b_ref, out_ref):
    a = agg_ref[:, :HID] + agg_ref[:, HID:] + htnp_ref[...]
    out_ref[...] = jnp.maximum(a * dinv_ref[...] + b_ref[...], 0.0)


_tc_last = pl.pallas_call(
    _tc_last_body,
    grid=(GRID,),
    in_specs=[
        pl.BlockSpec((BLK, NC * HID), lambda i: (i, 0)),
        pl.BlockSpec((BLK, HID), lambda i: (i, 0)),
        pl.BlockSpec((BLK, HID), lambda i: (i, 0)),
        pl.BlockSpec((1, HID), lambda i: (0, 0)),
    ],
    out_specs=pl.BlockSpec((BLK, HID), lambda i: (i, 0)),
    out_shape=jax.ShapeDtypeStruct((N, HID), jnp.float32),
)


def kernel(X, edge_index, W0, b0, W1, b1, W2, b2, W3, b3, W4, b4):
    ei = jnp.reshape(edge_index, (2, E_REAL_ROWS, CHUNK))
    pad_idx = jnp.arange(PAD_ROWS * CHUNK, dtype=jnp.int32)
    pad_blk = jnp.stack(
        [pad_idx % N, DUMMY + pad_idx % (N_PAD - N)]).reshape(2, PAD_ROWS, CHUNK)
    zeros16 = jnp.zeros((TILE_ROWS, 16), jnp.float32)
    ones16 = jnp.ones((CHUNK, 16), jnp.float32)
    zeros64 = jnp.zeros((TILE_ROWS, HID), jnp.float32)

    deg = _sc_degree(ei, pad_blk, ones16, zeros16).reshape(NC, N_PAD, 16)
    ht0 = _tc_first(X, W0)
    dinv, htn = _tc_scale(deg, deg, ht0)
    Ws = [W1, W2, W3, W4]
    bs = [b0.reshape(1, HID), b1.reshape(1, HID), b2.reshape(1, HID),
          b3.reshape(1, HID), b4.reshape(1, HID)]
    for l in range(4):
        agg = _sc_aggregate(htn, ei, pad_blk, zeros64)
        htn = _tc_mid(agg, htn, dinv, bs[l], Ws[l])
    agg = _sc_aggregate(htn, ei, pad_blk, zeros64)
    return _tc_last(agg, htn, dinv, bs[4])
